# Initial kernel scaffold; baseline (speedup 1.0000x reference)
import jax
import jax.numpy as jnp
from jax.experimental import pallas as pl


def kernel(hidden_states, expert_weights, w1, b1, w2, b2, top_experts):
    raise NotImplementedError("write your pallas kernel here")


# speedup vs baseline: 8.2733x; 8.2733x over previous
import functools

import jax
import jax.numpy as jnp
from jax.experimental import pallas as pl
from jax.experimental.pallas import tpu as pltpu

E = 64
HIDDEN = 1024
FFN = 2048
T = 8192
TM = 256
NT = T // TM
NP = NT + E - 1


def _moe_body(t_ids, g_ids, offs, nreal, x_ref, w1_ref, b1_ref, w2_ref,
              b2_ref, ew_ref, out_ref):
    i = pl.program_id(0)
    t = t_ids[i]
    g = g_ids[i]
    prev_t = t_ids[jnp.maximum(i - 1, 0)]
    first_visit = jnp.logical_or(i == 0, prev_t != t)

    @pl.when(first_visit)
    def _():
        out_ref[...] = jnp.zeros_like(out_ref)

    @pl.when(i < nreal[0])
    def _():
        row = t * TM + jax.lax.broadcasted_iota(jnp.int32, (TM, 1), 0)
        mask = jnp.logical_and(row >= offs[g], row < offs[g + 1])
        x = x_ref[...]
        fc1 = jnp.dot(x, w1_ref[0], preferred_element_type=jnp.float32)
        fc1 = fc1 + b1_ref[0]
        act = (0.5 * fc1 * (1.0 + jax.lax.erf(fc1 * 0.7071067811865476))
               ).astype(jnp.bfloat16)
        fc2 = jnp.dot(act, w2_ref[0], preferred_element_type=jnp.float32)
        fc2 = fc2 + b2_ref[0]
        val = fc2 * ew_ref[...]
        out_ref[...] = jnp.where(mask, val, out_ref[...])


def _grouped_ffn(t_ids, g_ids, offs, nreal, xs, w1, b1, w2, b2, ews):
    grid_spec = pltpu.PrefetchScalarGridSpec(
        num_scalar_prefetch=4,
        grid=(NP,),
        in_specs=[
            pl.BlockSpec((TM, HIDDEN), lambda i, T_, G, O, N: (T_[i], 0)),
            pl.BlockSpec((1, HIDDEN, FFN), lambda i, T_, G, O, N: (G[i], 0, 0)),
            pl.BlockSpec((1, 1, FFN), lambda i, T_, G, O, N: (G[i], 0, 0)),
            pl.BlockSpec((1, FFN, HIDDEN), lambda i, T_, G, O, N: (G[i], 0, 0)),
            pl.BlockSpec((1, 1, HIDDEN), lambda i, T_, G, O, N: (G[i], 0, 0)),
            pl.BlockSpec((TM, 1), lambda i, T_, G, O, N: (T_[i], 0)),
        ],
        out_specs=pl.BlockSpec((TM, HIDDEN), lambda i, T_, G, O, N: (T_[i], 0)),
    )
    return pl.pallas_call(
        _moe_body,
        grid_spec=grid_spec,
        out_shape=jax.ShapeDtypeStruct((T, HIDDEN), jnp.float32),
        compiler_params=pltpu.CompilerParams(
            dimension_semantics=("arbitrary",)),
    )(t_ids, g_ids, offs, nreal, xs, w1, b1, w2, b2, ews)


def _pair_metadata(offs, counts):
    g_arange = jnp.arange(E, dtype=jnp.int32)
    nonempty = counts > 0
    s_g = jnp.where(nonempty, offs[:-1] // TM, 0)
    l_g = jnp.where(nonempty, (offs[1:] - 1) // TM, -1)
    span = jnp.where(nonempty, l_g - s_g + 1, 0)
    pair_start = jnp.concatenate(
        [jnp.zeros((1,), jnp.int32), jnp.cumsum(span)[:-1].astype(jnp.int32)])
    nreal = jnp.sum(span).astype(jnp.int32)
    idx = jnp.arange(NP, dtype=jnp.int32)
    gid = (jnp.searchsorted(pair_start, idx, side="right").astype(jnp.int32)
           - 1)
    gid = jnp.clip(gid, 0, E - 1)
    tid = s_g[gid] + (idx - pair_start[gid])
    valid = idx < nreal
    sort_key = jnp.where(valid, tid, NT)
    order = jnp.argsort(sort_key, stable=True)
    gid = gid[order]
    tid = jnp.where(valid[order], tid[order], 0)
    last_g = gid[jnp.maximum(nreal - 1, 0)]
    last_t = tid[jnp.maximum(nreal - 1, 0)]
    gid = jnp.where(idx < nreal, gid, last_g)
    tid = jnp.where(idx < nreal, tid, last_t)
    return tid, gid, nreal[None]


def kernel(hidden_states, expert_weights, w1, b1, w2, b2, top_experts):
    hidden_shape = hidden_states.shape
    hs = hidden_states.reshape(-1, HIDDEN)
    top = top_experts.reshape(-1).astype(jnp.int32)
    ew = expert_weights.reshape(-1)

    counts = jnp.zeros((E,), jnp.int32).at[top].add(1)
    offs = jnp.concatenate(
        [jnp.zeros((1,), jnp.int32), jnp.cumsum(counts).astype(jnp.int32)])
    perm = jnp.argsort(top, stable=True).astype(jnp.int32)
    tid, gid, nreal = _pair_metadata(offs, counts)

    xs = hs[perm].astype(jnp.bfloat16)
    ews = ew[perm][:, None]

    out_sorted = _grouped_ffn(tid, gid, offs, nreal, xs,
                              w1.astype(jnp.bfloat16), b1[:, None, :],
                              w2.astype(jnp.bfloat16), b2[:, None, :], ews)

    out = jnp.zeros_like(hs).at[perm].set(out_sorted)
    return out.reshape(hidden_shape)
